# Initial kernel scaffold; baseline (speedup 1.0000x reference)
import jax
import jax.numpy as jnp
from jax.experimental import pallas as pl


def kernel(x, table, W, b):
    raise NotImplementedError("write your pallas kernel here")


# speedup vs baseline: 7.4589x; 7.4589x over previous
import functools

import jax
import jax.numpy as jnp
from jax import lax
from jax.experimental import pallas as pl
from jax.experimental.pallas import tpu as pltpu
from jax.experimental.pallas import tpu_sc as plsc

VOCAB = 1000000
EMBED = 16
BATCH = 16384
HIST = 200

NC = 2
NS = 16
L = 16
NW = NC * NS
ROWS_PER_W = BATCH // NW
GROUPS_PER_W = ROWS_PER_W // L
IDX_PER_GROUP = HIST * L
IDX_ROWS = IDX_PER_GROUP // 128

PROJ_M = VOCAB * EMBED // 128
PROJ_BLK = 1000


def _proj_body(t_ref, w_ref, o_ref):
    o_ref[...] = jnp.dot(t_ref[...], w_ref[...],
                         preferred_element_type=jnp.float32)


def _project(table_r, w8):
    return pl.pallas_call(
        _proj_body,
        grid=(PROJ_M // PROJ_BLK,),
        in_specs=[
            pl.BlockSpec((PROJ_BLK, 128), lambda i: (i, 0)),
            pl.BlockSpec((128, 8), lambda i: (0, 0)),
        ],
        out_specs=pl.BlockSpec((PROJ_BLK, 8), lambda i: (i, 0)),
        out_shape=jax.ShapeDtypeStruct((PROJ_M, 8), jnp.float32),
    )(table_r, w8)


def _sigmoid_round(logit):
    neg = logit < 0.0
    t = jnp.exp(jnp.where(neg, logit, -logit))
    sig = jnp.where(neg, t / (1.0 + t), 1.0 / (1.0 + t))
    y = sig * 10000.0
    r = (y + 8388608.0) - 8388608.0
    return r / 10000.0


def _pool_body(x1d_hbm, p_hbm, b_hbm, out_hbm, idx_v, vals_v, out_v, b_v,
               sem):
    cid = lax.axis_index("c")
    sid = lax.axis_index("s")
    wid = sid * NC + cid

    pltpu.sync_copy(b_hbm, b_v)
    bvec = b_v[...]
    base_lanes = lax.iota(jnp.int32, L) * HIST

    def group_body(g, carry):
        flat_base = (wid * GROUPS_PER_W + g) * IDX_PER_GROUP
        pltpu.sync_copy(x1d_hbm.at[pl.ds(flat_base, IDX_PER_GROUP)], idx_v)
        pltpu.async_copy(p_hbm.at[idx_v], vals_v, sem).wait()
        accs = [jnp.zeros((L,), jnp.float32) for _ in range(4)]
        for i in range(HIST):
            accs[i % 4] = accs[i % 4] + plsc.load_gather(
                vals_v, [base_lanes + i])
        logit = (accs[0] + accs[1]) + (accs[2] + accs[3]) + bvec
        out_v[pl.ds(g * L, L)] = _sigmoid_round(logit)
        return carry

    lax.fori_loop(0, GROUPS_PER_W, group_body, 0)
    pltpu.sync_copy(out_v, out_hbm.at[pl.ds(wid * ROWS_PER_W, ROWS_PER_W)])


@functools.cache
def _build_pool_kernel():
    return pl.kernel(
        _pool_body,
        out_type=jax.ShapeDtypeStruct((BATCH,), jnp.float32),
        mesh=plsc.VectorSubcoreMesh(core_axis_name="c", subcore_axis_name="s",
                                    num_cores=NC, num_subcores=NS),
        scratch_types=[
            pltpu.VMEM((IDX_PER_GROUP,), jnp.int32),
            pltpu.VMEM((IDX_PER_GROUP,), jnp.float32),
            pltpu.VMEM((ROWS_PER_W,), jnp.float32),
            pltpu.VMEM((L,), jnp.float32),
            pltpu.SemaphoreType.DMA,
        ],
        compiler_params=pltpu.CompilerParams(needs_layout_passes=False),
    )


def kernel(x, table, W, b):
    table_r = table.reshape(PROJ_M, 128)
    w8 = jnp.kron(jnp.eye(8, dtype=jnp.float32),
                  W.reshape(EMBED, 1) * (1.0 / HIST))
    p = _project(table_r, w8).reshape(VOCAB)
    x1d = x.reshape(BATCH * HIST)
    b16 = jnp.broadcast_to(b.astype(jnp.float32), (L,))
    out = _build_pool_kernel()(x1d, p, b16)
    return out.reshape(BATCH, 1)


# speedup vs baseline: 19.5416x; 2.6199x over previous
import functools

import jax
import jax.numpy as jnp
from jax import lax
from jax.experimental import pallas as pl
from jax.experimental.pallas import tpu as pltpu
from jax.experimental.pallas import tpu_sc as plsc

VOCAB = 1000000
EMBED = 16
BATCH = 16384
HIST = 200

NC = 2
NS = 16
L = 16
NW = NC * NS
ROWS_PER_W = BATCH // NW
CHUNK = 128
CHUNKS_PER_W = ROWS_PER_W // CHUNK
GROUPS_PER_CHUNK = CHUNK // L

PROJ_BLK = 8192


def _proj_body(t_ref, w_ref, o_ref):
    o_ref[...] = jnp.sum(t_ref[...] * w_ref[...], axis=0)


def _project(table_t, wcol):
    grid = (VOCAB + PROJ_BLK - 1) // PROJ_BLK
    return pl.pallas_call(
        _proj_body,
        grid=(grid,),
        in_specs=[
            pl.BlockSpec((EMBED, PROJ_BLK), lambda i: (0, i)),
            pl.BlockSpec((EMBED, 1), lambda i: (0, 0)),
        ],
        out_specs=pl.BlockSpec((PROJ_BLK,), lambda i: (i,)),
        out_shape=jax.ShapeDtypeStruct((VOCAB,), jnp.float32),
    )(table_t, wcol)


def _sigmoid_round(logit):
    neg = logit < 0.0
    t = jnp.exp(jnp.where(neg, logit, -logit))
    sig = jnp.where(neg, t / (1.0 + t), 1.0 / (1.0 + t))
    y = sig * 10000.0
    r = (y + 8388608.0) - 8388608.0
    return r / 10000.0


GROUPS_PER_W = ROWS_PER_W // L
IDX_PER_GROUP = HIST * L


def _pool_body(x1d_hbm, p_hbm, b_hbm, out_hbm, idx_v, vals_v, out_v, b_v,
               sem):
    cid = lax.axis_index("c")
    sid = lax.axis_index("s")
    wid = sid * NC + cid

    pltpu.sync_copy(b_hbm, b_v)
    bvec = b_v[...]
    base_lanes = lax.iota(jnp.int32, L) * HIST

    def group_body(g, carry):
        flat_base = (wid * GROUPS_PER_W + g) * IDX_PER_GROUP
        pltpu.sync_copy(x1d_hbm.at[pl.ds(flat_base, IDX_PER_GROUP)], idx_v)
        pltpu.async_copy(p_hbm.at[idx_v], vals_v, sem).wait()
        accs = [jnp.zeros((L,), jnp.float32) for _ in range(4)]
        for i in range(HIST):
            accs[i % 4] = accs[i % 4] + plsc.load_gather(
                vals_v, [base_lanes + i])
        logit = (accs[0] + accs[1]) + (accs[2] + accs[3]) + bvec
        out_v[pl.ds(g * L, L)] = _sigmoid_round(logit)
        return carry

    lax.fori_loop(0, GROUPS_PER_W, group_body, 0)
    pltpu.sync_copy(out_v, out_hbm.at[pl.ds(wid * ROWS_PER_W, ROWS_PER_W)])


@functools.cache
def _build_pool_kernel():
    return pl.kernel(
        _pool_body,
        out_type=jax.ShapeDtypeStruct((BATCH,), jnp.float32),
        mesh=plsc.VectorSubcoreMesh(core_axis_name="c", subcore_axis_name="s",
                                    num_cores=NC, num_subcores=NS),
        scratch_types=[
            pltpu.VMEM((IDX_PER_GROUP,), jnp.int32),
            pltpu.VMEM((IDX_PER_GROUP,), jnp.float32),
            pltpu.VMEM((ROWS_PER_W,), jnp.float32),
            pltpu.VMEM((L,), jnp.float32),
            pltpu.SemaphoreType.DMA,
        ],
        compiler_params=pltpu.CompilerParams(needs_layout_passes=False),
    )


def kernel(x, table, W, b):
    p = _project(table.T, W.reshape(EMBED, 1) * (1.0 / HIST))
    x1d = x.reshape(BATCH * HIST)
    b16 = jnp.broadcast_to(b.astype(jnp.float32), (L,))
    out = _build_pool_kernel()(x1d, p, b16)
    return out.reshape(BATCH, 1)


# speedup vs baseline: 28.5690x; 1.4620x over previous
import functools

import jax
import jax.numpy as jnp
from jax import lax
from jax.experimental import pallas as pl
from jax.experimental.pallas import tpu as pltpu
from jax.experimental.pallas import tpu_sc as plsc

VOCAB = 1000000
EMBED = 16
BATCH = 16384
HIST = 200

NC = 2
NS = 16
L = 16
NW = NC * NS
ROWS_PER_W = BATCH // NW
CHUNK = 128
CHUNKS_PER_W = ROWS_PER_W // CHUNK
GROUPS_PER_CHUNK = CHUNK // L

PROJ_BLK = 8192


def _proj_body(t_ref, w_ref, o_ref):
    o_ref[...] = jnp.sum(t_ref[...] * w_ref[...], axis=0)


def _project(table_t, wcol):
    grid = (VOCAB + PROJ_BLK - 1) // PROJ_BLK
    return pl.pallas_call(
        _proj_body,
        grid=(grid,),
        in_specs=[
            pl.BlockSpec((EMBED, PROJ_BLK), lambda i: (0, i)),
            pl.BlockSpec((EMBED, 1), lambda i: (0, 0)),
        ],
        out_specs=pl.BlockSpec((PROJ_BLK,), lambda i: (i,)),
        out_shape=jax.ShapeDtypeStruct((VOCAB,), jnp.float32),
    )(table_t, wcol)


def _sigmoid_round(logit):
    neg = logit < 0.0
    t = jnp.exp(jnp.where(neg, logit, -logit))
    sig = jnp.where(neg, t / (1.0 + t), 1.0 / (1.0 + t))
    y = sig * 10000.0
    r = (y + 8388608.0) - 8388608.0
    return r / 10000.0


GROUPS_PER_W = ROWS_PER_W // L
IDX_PER_GROUP = HIST * L


P_BNC = 8000
P_SUB = 8 * P_BNC


def _pool_body(x1d_hbm, p_hbm, b_hbm, out_hbm, idx_v, vals_v, out_v, b_v,
               p_sh, p_bnc, sem):
    cid = lax.axis_index("c")
    sid = lax.axis_index("s")
    wid = sid * NC + cid

    n_chunks = jnp.where(sid < NS - 1, 8, 5)

    def stage_body(j, carry):
        off = sid * P_SUB + j * P_BNC
        pltpu.sync_copy(p_hbm.at[pl.ds(off, P_BNC)], p_bnc)
        pltpu.sync_copy(p_bnc, p_sh.at[pl.ds(off, P_BNC)])
        return carry

    lax.fori_loop(0, n_chunks, stage_body, 0)

    pltpu.sync_copy(b_hbm, b_v)
    bvec = b_v[...]
    base_lanes = lax.iota(jnp.int32, L) * HIST
    plsc.subcore_barrier()

    def group_body(g, carry):
        flat_base = (wid * GROUPS_PER_W + g) * IDX_PER_GROUP
        pltpu.sync_copy(x1d_hbm.at[pl.ds(flat_base, IDX_PER_GROUP)], idx_v)
        pltpu.async_copy(p_sh.at[idx_v], vals_v, sem).wait()
        accs = [jnp.zeros((L,), jnp.float32) for _ in range(4)]
        for i in range(HIST):
            accs[i % 4] = accs[i % 4] + plsc.load_gather(
                vals_v, [base_lanes + i])
        logit = (accs[0] + accs[1]) + (accs[2] + accs[3]) + bvec
        out_v[pl.ds(g * L, L)] = _sigmoid_round(logit)
        return carry

    lax.fori_loop(0, GROUPS_PER_W, group_body, 0)
    pltpu.sync_copy(out_v, out_hbm.at[pl.ds(wid * ROWS_PER_W, ROWS_PER_W)])


@functools.cache
def _build_pool_kernel():
    return pl.kernel(
        _pool_body,
        out_type=jax.ShapeDtypeStruct((BATCH,), jnp.float32),
        mesh=plsc.VectorSubcoreMesh(core_axis_name="c", subcore_axis_name="s",
                                    num_cores=NC, num_subcores=NS),
        scratch_types=[
            pltpu.VMEM((IDX_PER_GROUP,), jnp.int32),
            pltpu.VMEM((IDX_PER_GROUP,), jnp.float32),
            pltpu.VMEM((ROWS_PER_W,), jnp.float32),
            pltpu.VMEM((L,), jnp.float32),
            pltpu.VMEM_SHARED((VOCAB,), jnp.float32),
            pltpu.VMEM((P_BNC,), jnp.float32),
            pltpu.SemaphoreType.DMA,
        ],
        compiler_params=pltpu.CompilerParams(needs_layout_passes=False),
    )


def kernel(x, table, W, b):
    p = _project(table.T, W.reshape(EMBED, 1) * (1.0 / HIST))
    x1d = x.reshape(BATCH * HIST)
    b16 = jnp.broadcast_to(b.astype(jnp.float32), (L,))
    out = _build_pool_kernel()(x1d, p, b16)
    return out.reshape(BATCH, 1)


# speedup vs baseline: 33.5387x; 1.1740x over previous
import functools

import jax
import jax.numpy as jnp
from jax import lax
from jax.experimental import pallas as pl
from jax.experimental.pallas import tpu as pltpu
from jax.experimental.pallas import tpu_sc as plsc

VOCAB = 1000000
EMBED = 16
BATCH = 16384
HIST = 200

NC = 2
NS = 16
L = 16
NW = NC * NS
ROWS_PER_W = BATCH // NW
CHUNK = 128
CHUNKS_PER_W = ROWS_PER_W // CHUNK
GROUPS_PER_CHUNK = CHUNK // L

PROJ_BLK = 8192


def _proj_body(t_ref, w_ref, o_ref):
    o_ref[...] = jnp.sum(t_ref[...] * w_ref[...], axis=0)


def _project(table_t, wcol):
    grid = (VOCAB + PROJ_BLK - 1) // PROJ_BLK
    return pl.pallas_call(
        _proj_body,
        grid=(grid,),
        in_specs=[
            pl.BlockSpec((EMBED, PROJ_BLK), lambda i: (0, i)),
            pl.BlockSpec((EMBED, 1), lambda i: (0, 0)),
        ],
        out_specs=pl.BlockSpec((PROJ_BLK,), lambda i: (i,)),
        out_shape=jax.ShapeDtypeStruct((VOCAB,), jnp.float32),
    )(table_t, wcol)


def _sigmoid_round(logit):
    neg = logit < 0.0
    t = jnp.exp(jnp.where(neg, logit, -logit))
    sig = jnp.where(neg, t / (1.0 + t), 1.0 / (1.0 + t))
    y = sig * 10000.0
    r = (y + 8388608.0) - 8388608.0
    return r / 10000.0


GROUPS_PER_W = ROWS_PER_W // L
IDX_PER_GROUP = HIST * L


P_BNC = 8000
P_SUB = 8 * P_BNC


def _pool_body(x1d_hbm, p_hbm, b_hbm, out_hbm, idx_v, idx_w, vals_v, vals_w,
               out_v, b_v, p_sh, p_bnc, semg0, semg1, semi0, semi1):
    cid = lax.axis_index("c")
    sid = lax.axis_index("s")
    wid = sid * NC + cid

    n_chunks = jnp.where(sid < NS - 1, 8, 5)

    def stage_body(j, carry):
        off = sid * P_SUB + j * P_BNC
        pltpu.sync_copy(p_hbm.at[pl.ds(off, P_BNC)], p_bnc)
        pltpu.sync_copy(p_bnc, p_sh.at[pl.ds(off, P_BNC)])
        return carry

    lax.fori_loop(0, n_chunks, stage_body, 0)

    pltpu.sync_copy(b_hbm, b_v)
    bvec = b_v[...]
    base_lanes = lax.iota(jnp.int32, L) * HIST
    plsc.subcore_barrier()

    idx_bufs = (idx_v, idx_w)
    val_bufs = (vals_v, vals_w)
    gsems = (semg0, semg1)
    isems = (semi0, semi1)
    g_base = wid * GROUPS_PER_W

    def idx_src(g):
        return x1d_hbm.at[pl.ds((g_base + g) * IDX_PER_GROUP, IDX_PER_GROUP)]

    def reduce_store(g, vals):
        accs = [jnp.zeros((L,), jnp.float32) for _ in range(4)]
        for i in range(HIST):
            accs[i % 4] = accs[i % 4] + plsc.load_gather(
                vals, [base_lanes + i])
        logit = (accs[0] + accs[1]) + (accs[2] + accs[3]) + bvec
        out_v[pl.ds(g * L, L)] = _sigmoid_round(logit)

    pltpu.sync_copy(idx_src(0), idx_bufs[0])
    pltpu.async_copy(p_sh.at[idx_bufs[0]], val_bufs[0], gsems[0])
    pltpu.async_copy(idx_src(1), idx_bufs[1], isems[1])

    def pair_body(i, carry):
        for b in (0, 1):
            g = 2 * i + b
            nb = 1 - b

            @pl.when(g + 1 < GROUPS_PER_W)
            def _():
                pltpu.make_async_copy(idx_src(g + 1), idx_bufs[nb],
                                      isems[nb]).wait()
                pltpu.async_copy(p_sh.at[idx_bufs[nb]], val_bufs[nb],
                                 gsems[nb])

            pltpu.make_async_copy(p_sh.at[idx_bufs[b]], val_bufs[b],
                                  gsems[b]).wait()

            @pl.when(g + 2 < GROUPS_PER_W)
            def _():
                pltpu.async_copy(idx_src(g + 2), idx_bufs[b], isems[b])

            reduce_store(g, val_bufs[b])
        return carry

    lax.fori_loop(0, GROUPS_PER_W // 2, pair_body, 0)
    pltpu.sync_copy(out_v, out_hbm.at[pl.ds(wid * ROWS_PER_W, ROWS_PER_W)])


@functools.cache
def _build_pool_kernel():
    return pl.kernel(
        _pool_body,
        out_type=jax.ShapeDtypeStruct((BATCH,), jnp.float32),
        mesh=plsc.VectorSubcoreMesh(core_axis_name="c", subcore_axis_name="s",
                                    num_cores=NC, num_subcores=NS),
        scratch_types=[
            pltpu.VMEM((IDX_PER_GROUP,), jnp.int32),
            pltpu.VMEM((IDX_PER_GROUP,), jnp.int32),
            pltpu.VMEM((IDX_PER_GROUP,), jnp.float32),
            pltpu.VMEM((IDX_PER_GROUP,), jnp.float32),
            pltpu.VMEM((ROWS_PER_W,), jnp.float32),
            pltpu.VMEM((L,), jnp.float32),
            pltpu.VMEM_SHARED((VOCAB,), jnp.float32),
            pltpu.VMEM((P_BNC,), jnp.float32),
            pltpu.SemaphoreType.DMA,
            pltpu.SemaphoreType.DMA,
            pltpu.SemaphoreType.DMA,
            pltpu.SemaphoreType.DMA,
        ],
        compiler_params=pltpu.CompilerParams(needs_layout_passes=False),
    )


def kernel(x, table, W, b):
    p = _project(table.T, W.reshape(EMBED, 1) * (1.0 / HIST))
    x1d = x.reshape(BATCH * HIST)
    b16 = jnp.broadcast_to(b.astype(jnp.float32), (L,))
    out = _build_pool_kernel()(x1d, p, b16)
    return out.reshape(BATCH, 1)


# speedup vs baseline: 44.5876x; 1.3294x over previous
import functools

import jax
import jax.numpy as jnp
from jax import lax
from jax.experimental import pallas as pl
from jax.experimental.pallas import tpu as pltpu
from jax.experimental.pallas import tpu_sc as plsc

VOCAB = 1000000
EMBED = 16
BATCH = 16384
HIST = 200

NC = 2
NS = 16
L = 16
NW = NC * NS
ROWS_PER_W = BATCH // NW
CHUNK = 128
CHUNKS_PER_W = ROWS_PER_W // CHUNK
GROUPS_PER_CHUNK = CHUNK // L

PROJ_BLK = 32768


def _proj_body(t_ref, w_ref, o_ref):
    o_ref[...] = jnp.sum(t_ref[...] * w_ref[...], axis=0)


def _project(table_t, wcol):
    grid = (VOCAB + PROJ_BLK - 1) // PROJ_BLK
    return pl.pallas_call(
        _proj_body,
        grid=(grid,),
        in_specs=[
            pl.BlockSpec((EMBED, PROJ_BLK), lambda i: (0, i)),
            pl.BlockSpec((EMBED, 1), lambda i: (0, 0)),
        ],
        out_specs=pl.BlockSpec((PROJ_BLK,), lambda i: (i,)),
        out_shape=jax.ShapeDtypeStruct((VOCAB,), jnp.float32),
    )(table_t, wcol)


def _sigmoid_round(logit):
    neg = logit < 0.0
    t = jnp.exp(jnp.where(neg, logit, -logit))
    sig = jnp.where(neg, t / (1.0 + t), 1.0 / (1.0 + t))
    y = sig * 10000.0
    r = (y + 8388608.0) - 8388608.0
    return r / 10000.0


GROUPS_PER_W = ROWS_PER_W // L
IDX_PER_GROUP = HIST * L


P_BNC = 8000
P_SUB = 8 * P_BNC


def _pool_body(x1d_hbm, p_hbm, b_hbm, out_hbm, idx_v, idx_w, vals_v, vals_w,
               out_v, b_v, p_sh, p_bnc, semg0, semg1, semi0, semi1):
    cid = lax.axis_index("c")
    sid = lax.axis_index("s")
    wid = sid * NC + cid

    n_chunks = jnp.where(sid < NS - 1, 8, 5)

    def stage_body(j, carry):
        off = sid * P_SUB + j * P_BNC
        pltpu.sync_copy(p_hbm.at[pl.ds(off, P_BNC)], p_bnc)
        pltpu.sync_copy(p_bnc, p_sh.at[pl.ds(off, P_BNC)])
        return carry

    lax.fori_loop(0, n_chunks, stage_body, 0)

    pltpu.sync_copy(b_hbm, b_v)
    bvec = b_v[...]
    base_lanes = lax.iota(jnp.int32, L) * HIST
    plsc.subcore_barrier()

    idx_bufs = (idx_v, idx_w)
    val_bufs = (vals_v, vals_w)
    gsems = (semg0, semg1)
    isems = (semi0, semi1)
    g_base = wid * GROUPS_PER_W

    def idx_src(g):
        return x1d_hbm.at[pl.ds((g_base + g) * IDX_PER_GROUP, IDX_PER_GROUP)]

    def reduce_store(g, vals):
        accs = [jnp.zeros((L,), jnp.float32) for _ in range(4)]
        for i in range(HIST):
            accs[i % 4] = accs[i % 4] + plsc.load_gather(
                vals, [base_lanes + i])
        logit = (accs[0] + accs[1]) + (accs[2] + accs[3]) + bvec
        out_v[pl.ds(g * L, L)] = _sigmoid_round(logit)

    pltpu.sync_copy(idx_src(0), idx_bufs[0])
    pltpu.async_copy(p_sh.at[idx_bufs[0]], val_bufs[0], gsems[0])
    pltpu.async_copy(idx_src(1), idx_bufs[1], isems[1])

    def pair_body(i, carry):
        for b in (0, 1):
            g = 2 * i + b
            nb = 1 - b

            @pl.when(g + 1 < GROUPS_PER_W)
            def _():
                pltpu.make_async_copy(idx_src(g + 1), idx_bufs[nb],
                                      isems[nb]).wait()
                pltpu.async_copy(p_sh.at[idx_bufs[nb]], val_bufs[nb],
                                 gsems[nb])

            pltpu.make_async_copy(p_sh.at[idx_bufs[b]], val_bufs[b],
                                  gsems[b]).wait()

            @pl.when(g + 2 < GROUPS_PER_W)
            def _():
                pltpu.async_copy(idx_src(g + 2), idx_bufs[b], isems[b])

            reduce_store(g, val_bufs[b])
        return carry

    lax.fori_loop(0, GROUPS_PER_W // 2, pair_body, 0)
    pltpu.sync_copy(out_v, out_hbm.at[pl.ds(wid * ROWS_PER_W, ROWS_PER_W)])


@functools.cache
def _build_pool_kernel():
    return pl.kernel(
        _pool_body,
        out_type=jax.ShapeDtypeStruct((BATCH,), jnp.float32),
        mesh=plsc.VectorSubcoreMesh(core_axis_name="c", subcore_axis_name="s",
                                    num_cores=NC, num_subcores=NS),
        scratch_types=[
            pltpu.VMEM((IDX_PER_GROUP,), jnp.int32),
            pltpu.VMEM((IDX_PER_GROUP,), jnp.int32),
            pltpu.VMEM((IDX_PER_GROUP,), jnp.float32),
            pltpu.VMEM((IDX_PER_GROUP,), jnp.float32),
            pltpu.VMEM((ROWS_PER_W,), jnp.float32),
            pltpu.VMEM((L,), jnp.float32),
            pltpu.VMEM_SHARED((VOCAB,), jnp.float32),
            pltpu.VMEM((P_BNC,), jnp.float32),
            pltpu.SemaphoreType.DMA,
            pltpu.SemaphoreType.DMA,
            pltpu.SemaphoreType.DMA,
            pltpu.SemaphoreType.DMA,
        ],
        compiler_params=pltpu.CompilerParams(needs_layout_passes=False),
    )


def kernel(x, table, W, b):
    p = _project(table.T, W.reshape(EMBED, 1) * (1.0 / HIST))
    x1d = x.reshape(BATCH * HIST)
    b16 = jnp.broadcast_to(b.astype(jnp.float32), (L,))
    out = _build_pool_kernel()(x1d, p, b16)
    return out.reshape(BATCH, 1)


# speedup vs baseline: 47.4472x; 1.0641x over previous
import functools

import jax
import jax.numpy as jnp
from jax import lax
from jax.experimental import pallas as pl
from jax.experimental.pallas import tpu as pltpu
from jax.experimental.pallas import tpu_sc as plsc

VOCAB = 1000000
EMBED = 16
BATCH = 16384
HIST = 200

NC = 2
NS = 16
L = 16
NW = NC * NS
ROWS_PER_W = BATCH // NW
CHUNK = 128
CHUNKS_PER_W = ROWS_PER_W // CHUNK
GROUPS_PER_CHUNK = CHUNK // L

PROJ_BLK = 131072


def _proj_body(t_ref, w_ref, o_ref):
    o_ref[...] = jnp.sum(t_ref[...] * w_ref[...], axis=0)


def _project(table_t, wcol):
    grid = (VOCAB + PROJ_BLK - 1) // PROJ_BLK
    return pl.pallas_call(
        _proj_body,
        grid=(grid,),
        in_specs=[
            pl.BlockSpec((EMBED, PROJ_BLK), lambda i: (0, i)),
            pl.BlockSpec((EMBED, 1), lambda i: (0, 0)),
        ],
        out_specs=pl.BlockSpec((PROJ_BLK,), lambda i: (i,)),
        out_shape=jax.ShapeDtypeStruct((VOCAB,), jnp.float32),
    )(table_t, wcol)


def _sigmoid_round(logit):
    neg = logit < 0.0
    t = jnp.exp(jnp.where(neg, logit, -logit))
    sig = jnp.where(neg, t / (1.0 + t), 1.0 / (1.0 + t))
    y = sig * 10000.0
    r = (y + 8388608.0) - 8388608.0
    return r / 10000.0


GROUPS_PER_W = ROWS_PER_W // L
IDX_PER_GROUP = HIST * L


P_BNC = 8000
P_SUB = 8 * P_BNC


def _pool_body(x1d_hbm, p_hbm, b_hbm, out_hbm, idx_v, idx_w, vals_v, vals_w,
               out_v, b_v, p_sh, p_bnc, semg0, semg1, semi0, semi1):
    cid = lax.axis_index("c")
    sid = lax.axis_index("s")
    wid = sid * NC + cid

    n_chunks = jnp.where(sid < NS - 1, 8, 5)

    def stage_body(j, carry):
        off = sid * P_SUB + j * P_BNC
        pltpu.sync_copy(p_hbm.at[pl.ds(off, P_BNC)], p_bnc)
        pltpu.sync_copy(p_bnc, p_sh.at[pl.ds(off, P_BNC)])
        return carry

    lax.fori_loop(0, n_chunks, stage_body, 0)

    pltpu.sync_copy(b_hbm, b_v)
    bvec = b_v[...]
    base_lanes = lax.iota(jnp.int32, L) * HIST
    plsc.subcore_barrier()

    idx_bufs = (idx_v, idx_w)
    val_bufs = (vals_v, vals_w)
    gsems = (semg0, semg1)
    isems = (semi0, semi1)
    g_base = wid * GROUPS_PER_W

    def idx_src(g):
        return x1d_hbm.at[pl.ds((g_base + g) * IDX_PER_GROUP, IDX_PER_GROUP)]

    def reduce_store(g, vals):
        accs = [jnp.zeros((L,), jnp.float32) for _ in range(4)]
        for i in range(HIST):
            accs[i % 4] = accs[i % 4] + plsc.load_gather(
                vals, [base_lanes + i])
        logit = (accs[0] + accs[1]) + (accs[2] + accs[3]) + bvec
        out_v[pl.ds(g * L, L)] = _sigmoid_round(logit)

    pltpu.sync_copy(idx_src(0), idx_bufs[0])
    pltpu.async_copy(p_sh.at[idx_bufs[0]], val_bufs[0], gsems[0])
    pltpu.async_copy(idx_src(1), idx_bufs[1], isems[1])

    def pair_body(i, carry):
        for b in (0, 1):
            g = 2 * i + b
            nb = 1 - b

            @pl.when(g + 1 < GROUPS_PER_W)
            def _():
                pltpu.make_async_copy(idx_src(g + 1), idx_bufs[nb],
                                      isems[nb]).wait()
                pltpu.async_copy(p_sh.at[idx_bufs[nb]], val_bufs[nb],
                                 gsems[nb])

            pltpu.make_async_copy(p_sh.at[idx_bufs[b]], val_bufs[b],
                                  gsems[b]).wait()

            @pl.when(g + 2 < GROUPS_PER_W)
            def _():
                pltpu.async_copy(idx_src(g + 2), idx_bufs[b], isems[b])

            reduce_store(g, val_bufs[b])
        return carry

    lax.fori_loop(0, GROUPS_PER_W // 2, pair_body, 0)
    pltpu.sync_copy(out_v, out_hbm.at[pl.ds(wid * ROWS_PER_W, ROWS_PER_W)])


@functools.cache
def _build_pool_kernel():
    return pl.kernel(
        _pool_body,
        out_type=jax.ShapeDtypeStruct((BATCH,), jnp.float32),
        mesh=plsc.VectorSubcoreMesh(core_axis_name="c", subcore_axis_name="s",
                                    num_cores=NC, num_subcores=NS),
        scratch_types=[
            pltpu.VMEM((IDX_PER_GROUP,), jnp.int32),
            pltpu.VMEM((IDX_PER_GROUP,), jnp.int32),
            pltpu.VMEM((IDX_PER_GROUP,), jnp.float32),
            pltpu.VMEM((IDX_PER_GROUP,), jnp.float32),
            pltpu.VMEM((ROWS_PER_W,), jnp.float32),
            pltpu.VMEM((L,), jnp.float32),
            pltpu.VMEM_SHARED((VOCAB,), jnp.float32),
            pltpu.VMEM((P_BNC,), jnp.float32),
            pltpu.SemaphoreType.DMA,
            pltpu.SemaphoreType.DMA,
            pltpu.SemaphoreType.DMA,
            pltpu.SemaphoreType.DMA,
        ],
        compiler_params=pltpu.CompilerParams(needs_layout_passes=False),
    )


def kernel(x, table, W, b):
    p = _project(table.T, W.reshape(EMBED, 1) * (1.0 / HIST))
    x1d = x.reshape(BATCH * HIST)
    b16 = jnp.broadcast_to(b.astype(jnp.float32), (L,))
    out = _build_pool_kernel()(x1d, p, b16)
    return out.reshape(BATCH, 1)
